# baseline (speedup 1.0000x reference)
import functools

import jax
import jax.numpy as jnp
from jax import lax
from jax.experimental import pallas as pl
from jax.experimental.pallas import tpu as pltpu
from jax.experimental.pallas import tpu_sc as plsc

B = 4096
DENSE = 8
N_SPARSE = 10
SPARSE_VOCAB = 100000
SEQ_VOCAB = 1000000
D = 32
MAXLEN = 50
M_ITEMS = 100

NC = 2
NS = 16
NW = NC * NS

SEQ_ROWS = B * MAXLEN
SEQ_PER_W = SEQ_ROWS // NW
SEQ_CHUNK = 1600
SPARSE_ROWS = B * N_SPARSE
SPARSE_PER_W = SPARSE_ROWS // NW
ITEM_PAD = 128
LINES = 4

QS_SEQ = 262144
QS_SP = 32768
TBLK = 4096
SEQ_LASTB = (SEQ_VOCAB + TBLK - 1) // TBLK - 1
SP_LASTB = (SPARSE_VOCAB + TBLK - 1) // TBLK - 1

GQ = 4
BL = B // GQ


def _transpose_seq_body(t0, t1, t2, t3, out_ref):
    for q, t in enumerate((t0, t1, t2, t3)):
        out_ref[:, q * D:(q + 1) * D] = jnp.transpose(t[...])


_transpose_seq = pl.pallas_call(
    _transpose_seq_body,
    grid=(QS_SEQ // TBLK,),
    in_specs=[
        pl.BlockSpec(
            (D, TBLK),
            lambda i, q=q: (0, jnp.minimum(i + q * (QS_SEQ // TBLK),
                                           SEQ_LASTB)))
        for q in range(LINES)
    ],
    out_specs=pl.BlockSpec((TBLK, 128), lambda i: (i, 0)),
    out_shape=jax.ShapeDtypeStruct((QS_SEQ, 128), jnp.float32),
)


def _transpose_sp_body(t0, t1, t2, t3, out_ref):
    for q, t in enumerate((t0, t1, t2, t3)):
        out_ref[:, q * D:(q + 1) * D] = jnp.transpose(t[...])


_transpose_sp = pl.pallas_call(
    _transpose_sp_body,
    grid=(N_SPARSE, QS_SP // TBLK),
    in_specs=[
        pl.BlockSpec(
            (D, TBLK),
            lambda i, j, q=q: (i, jnp.minimum(j + q * (QS_SP // TBLK),
                                              SP_LASTB)))
        for q in range(LINES)
    ],
    out_specs=pl.BlockSpec((TBLK, 128),
                           lambda i, j: (i * (QS_SP // TBLK) + j, 0)),
    out_shape=jax.ShapeDtypeStruct((N_SPARSE * QS_SP, 128), jnp.float32),
)


def _sc_gather_body(seq_idx, sparse_idx, item_idx, table_seq, table_sparse,
                    seq_out, sparse_out, item_out,
                    idx_v, rows_v, sidx_v, srows_v, iidx_v, irows_v, sem):
    wid = lax.axis_index("s") * NC + lax.axis_index("c")
    base = wid * SEQ_PER_W
    for ci in range(SEQ_PER_W // SEQ_CHUNK):
        off = base + ci * SEQ_CHUNK
        pltpu.sync_copy(seq_idx.at[pl.ds(off, SEQ_CHUNK)], idx_v)
        pltpu.async_copy(table_seq.at[idx_v], rows_v, sem).wait()
        pltpu.sync_copy(rows_v, seq_out.at[pl.ds(off, SEQ_CHUNK)])
    sbase = wid * SPARSE_PER_W
    pltpu.sync_copy(sparse_idx.at[pl.ds(sbase, SPARSE_PER_W)], sidx_v)
    pltpu.async_copy(table_sparse.at[sidx_v], srows_v, sem).wait()
    pltpu.sync_copy(srows_v, sparse_out.at[pl.ds(sbase, SPARSE_PER_W)])
    @pl.when(wid == 0)
    def _():
        pltpu.sync_copy(item_idx.at[pl.ds(0, ITEM_PAD)], iidx_v)
        pltpu.async_copy(table_seq.at[iidx_v], irows_v, sem).wait()
        pltpu.sync_copy(irows_v, item_out.at[pl.ds(0, ITEM_PAD)])


@functools.lru_cache(maxsize=1)
def _get_sc_gather():
  return pl.kernel(
    _sc_gather_body,
    mesh=plsc.VectorSubcoreMesh(core_axis_name="c", subcore_axis_name="s"),
    out_type=[
        jax.ShapeDtypeStruct((SEQ_ROWS, D), jnp.float32),
        jax.ShapeDtypeStruct((SPARSE_ROWS, D), jnp.float32),
        jax.ShapeDtypeStruct((ITEM_PAD, D), jnp.float32),
    ],
    scratch_types=[
        pltpu.VMEM((SEQ_CHUNK,), jnp.int32),
        pltpu.VMEM((SEQ_CHUNK, D), jnp.float32),
        pltpu.VMEM((SPARSE_PER_W,), jnp.int32),
        pltpu.VMEM((SPARSE_PER_W, D), jnp.float32),
        pltpu.VMEM((ITEM_PAD,), jnp.int32),
        pltpu.VMEM((ITEM_PAD, D), jnp.float32),
        pltpu.SemaphoreType.DMA,
    ],
    compiler_params=pltpu.CompilerParams(use_tc_tiling_on_sc=False),
  )


BB = 256
GB = BB // GQ
G = B // BB

IW1, IW2, IW3, IF1M, IF2M, IF1S, IF2S = 0, 1, 2, 3, 4, 5, 15


def _tc_dense_body(seq_ref, dense_ref, sparse_ref, item_ref, sel_ref,
                   wpk_ref, wd_ref, wb_ref, out_ref):
    ri = lax.broadcasted_iota(jnp.int32, (128, 128), 0) // D
    ci = lax.broadcasted_iota(jnp.int32, (128, 128), 1) // D
    gones = (ri == ci).astype(jnp.float32)

    S = seq_ref[...]
    m_s = jnp.mean(S, axis=0)
    m_t = S[MAXLEN - 1]
    c = m_s @ wpk_ref[IW2] + m_t @ wpk_ref[IW3] + wb_ref[1:2]
    S2 = S.reshape(MAXLEN * GB, 128)
    E = (S2 @ wpk_ref[IW1]).reshape(MAXLEN, GB, 128) + c[None, :, :]
    att = jax.nn.sigmoid(E)
    aw = att * wb_ref[0:1][None, :, :]
    alpha = (aw.reshape(MAXLEN * GB, 128) @ gones).reshape(MAXLEN, GB, 128)
    m_a = jnp.sum(alpha * S, axis=0)

    xd = dense_ref[...]
    acc1 = m_a @ wpk_ref[IF1M] + xd @ wd_ref[0] + wb_ref[2:3]
    acc2 = m_t @ wpk_ref[IF2M] + xd @ wd_ref[1] + wb_ref[3:4]
    for i in range(N_SPARSE):
        xi = sparse_ref[i]
        acc1 = acc1 + xi @ wpk_ref[IF1S + i]
        acc2 = acc2 + xi @ wpk_ref[IF2S + i]
    p4 = jnp.tanh(acc1) * jnp.tanh(acc2)

    p = sel_ref[0] @ p4[:, 0:D]
    for q in range(1, GQ):
        p = p + sel_ref[q] @ p4[:, q * D:(q + 1) * D]
    z = lax.dot_general(p, item_ref[...], (((1,), (1,)), ((), ())))
    z = z[:, :M_ITEMS]
    z = z - jnp.max(z, axis=-1, keepdims=True)
    ez = jnp.exp(z)
    out_ref[...] = ez / jnp.sum(ez, axis=-1, keepdims=True)


def _full_spec(shape):
    return pl.BlockSpec(shape, lambda i: tuple(0 for _ in shape))


_TC_IN_SPECS = [
        pl.BlockSpec((MAXLEN, GB, 128), lambda i: (0, i, 0)),
        pl.BlockSpec((GB, GQ * DENSE), lambda i: (i, 0)),
        pl.BlockSpec((N_SPARSE, GB, 128), lambda i: (0, i, 0)),
        _full_spec((ITEM_PAD, D)),
        _full_spec((GQ, BB, GB)),
        _full_spec((25, 128, 128)),
        _full_spec((2, GQ * DENSE, 128)),
        _full_spec((GQ, 128)),
]

_tc_dense = pl.pallas_call(
    _tc_dense_body,
    grid=(G,),
    in_specs=_TC_IN_SPECS,
    out_specs=pl.BlockSpec((BB, M_ITEMS), lambda i: (i, 0)),
    out_shape=jax.ShapeDtypeStruct((B, M_ITEMS), jnp.float32),
)


@jax.jit
def kernel(dense_inputs, sparse_inputs, seq_inputs, item_pooling, table_sparse,
           table_seq, W0, W1, W2, W3, b, ffn1_W, ffn1_b, ffn2_W, ffn2_b):
    tseqT = table_seq.T
    tspT = jnp.transpose(table_sparse, (0, 2, 1)).reshape(
        N_SPARSE * D, SPARSE_VOCAB)
    tseq_g = _transpose_seq(tseqT, tseqT, tseqT, tseqT).reshape(-1, D)
    tsp_g = _transpose_sp(tspT, tspT, tspT, tspT).reshape(-1, D)

    fseq = seq_inputs[:, 0, :].T
    fsp = sparse_inputs.T
    sp_base = (jnp.arange(N_SPARSE, dtype=jnp.int32) * QS_SP)[:, None]
    fit = jnp.concatenate(
        [item_pooling[:, 0], jnp.zeros((ITEM_PAD - M_ITEMS,), jnp.int32)])

    seq_flat, sparse_flat, item_embed = _get_sc_gather()(
        (LINES * (fseq % QS_SEQ) + fseq // QS_SEQ).reshape(-1),
        (LINES * (fsp % QS_SP + sp_base) + fsp // QS_SP).reshape(-1),
        LINES * (fit % QS_SEQ) + fit // QS_SEQ,
        tseq_g, tsp_g)

    seq_pk = seq_flat.reshape(MAXLEN, BL, 128)
    sparse_pk = sparse_flat.reshape(N_SPARSE, BL, 128)
    dense_pk = dense_inputs.reshape(BL, GQ * DENSE)

    wsm = jnp.concatenate([
        W1[None], W2[None], W3[None], ffn1_W[None, :D], ffn2_W[None, :D],
        ffn1_W[D + DENSE:].reshape(N_SPARSE, D, D),
        ffn2_W[D + DENSE:].reshape(N_SPARSE, D, D)])
    ri = jnp.arange(GQ * D)[:, None] // D
    ci = jnp.arange(GQ * D)[None, :] // D
    wpk = jnp.tile(wsm, (1, GQ, GQ)) * (ri == ci)[None]
    wdsm = jnp.stack([ffn1_W[D:D + DENSE], ffn2_W[D:D + DENSE]])
    rd = jnp.arange(GQ * DENSE)[:, None] // DENSE
    cd = jnp.arange(GQ * D)[None, :] // D
    wd = jnp.tile(wdsm, (1, GQ, GQ)) * (rd == cd)[None]
    wb = jnp.tile(jnp.stack([W0[:, 0], b, ffn1_b, ffn2_b]), (1, GQ))
    rows = jnp.arange(BB)
    cols = jnp.arange(GB)
    sel = jnp.stack([
        (rows[:, None] == cols[None, :] * GQ + q).astype(jnp.float32)
        for q in range(GQ)])

    return _tc_dense(seq_pk, dense_pk, sparse_pk, item_embed, sel,
                     wpk, wd, wb)


# speedup vs baseline: 2.1040x; 2.1040x over previous
import functools

import jax
import jax.numpy as jnp
from jax import lax
from jax.experimental import pallas as pl
from jax.experimental.pallas import tpu as pltpu
from jax.experimental.pallas import tpu_sc as plsc

B = 4096
DENSE = 8
N_SPARSE = 10
SPARSE_VOCAB = 100000
SEQ_VOCAB = 1000000
D = 32
MAXLEN = 50
M_ITEMS = 100

NC = 2
NS = 16
NW = NC * NS

SEQ_ROWS = B * MAXLEN
SEQ_PER_W = SEQ_ROWS // NW
SEQ_CHUNK = 1600
SPARSE_ROWS = B * N_SPARSE
SPARSE_PER_W = SPARSE_ROWS // NW
ITEM_PAD = 128
LINES = 4

QS_SEQ = 262144
QS_SP = 32768
TBLK = 4096
SEQ_LASTB = (SEQ_VOCAB + TBLK - 1) // TBLK - 1
SP_LASTB = (SPARSE_VOCAB + TBLK - 1) // TBLK - 1

GQ = 4
BL = B // GQ


def _transpose_seq_body(t0, t1, t2, t3, out_ref):
    x = jnp.concatenate([t0[...], t1[...], t2[...], t3[...]], axis=0)
    out_ref[...] = jnp.transpose(x)


_transpose_seq = pl.pallas_call(
    _transpose_seq_body,
    grid=(QS_SEQ // TBLK,),
    in_specs=[
        pl.BlockSpec(
            (D, TBLK),
            lambda i, q=q: (0, jnp.minimum(i + q * (QS_SEQ // TBLK),
                                           SEQ_LASTB)))
        for q in range(LINES)
    ],
    out_specs=pl.BlockSpec((TBLK, 128), lambda i: (i, 0)),
    out_shape=jax.ShapeDtypeStruct((QS_SEQ, 128), jnp.float32),
)


def _transpose_sp_body(t0, t1, t2, t3, out_ref):
    x = jnp.concatenate([t0[...], t1[...], t2[...], t3[...]], axis=0)
    out_ref[...] = jnp.transpose(x)


_transpose_sp = pl.pallas_call(
    _transpose_sp_body,
    grid=(N_SPARSE, QS_SP // TBLK),
    in_specs=[
        pl.BlockSpec(
            (D, TBLK),
            lambda i, j, q=q: (i, jnp.minimum(j + q * (QS_SP // TBLK),
                                              SP_LASTB)))
        for q in range(LINES)
    ],
    out_specs=pl.BlockSpec((TBLK, 128),
                           lambda i, j: (i * (QS_SP // TBLK) + j, 0)),
    out_shape=jax.ShapeDtypeStruct((N_SPARSE * QS_SP, 128), jnp.float32),
)


def _sc_gather_body(seq_idx, sparse_idx, item_idx, table_seq, table_sparse,
                    seq_out, sparse_out, item_out,
                    idx_v, rows_v, sidx_v, srows_v, iidx_v, irows_v, sem):
    wid = lax.axis_index("s") * NC + lax.axis_index("c")
    base = wid * SEQ_PER_W
    for ci in range(SEQ_PER_W // SEQ_CHUNK):
        off = base + ci * SEQ_CHUNK
        pltpu.sync_copy(seq_idx.at[pl.ds(off, SEQ_CHUNK)], idx_v)
        pltpu.async_copy(table_seq.at[idx_v], rows_v, sem).wait()
        pltpu.sync_copy(rows_v, seq_out.at[pl.ds(off, SEQ_CHUNK)])
    sbase = wid * SPARSE_PER_W
    pltpu.sync_copy(sparse_idx.at[pl.ds(sbase, SPARSE_PER_W)], sidx_v)
    pltpu.async_copy(table_sparse.at[sidx_v], srows_v, sem).wait()
    pltpu.sync_copy(srows_v, sparse_out.at[pl.ds(sbase, SPARSE_PER_W)])
    @pl.when(wid == 0)
    def _():
        pltpu.sync_copy(item_idx.at[pl.ds(0, ITEM_PAD)], iidx_v)
        pltpu.async_copy(table_seq.at[iidx_v], irows_v, sem).wait()
        pltpu.sync_copy(irows_v, item_out.at[pl.ds(0, ITEM_PAD)])


@functools.lru_cache(maxsize=1)
def _get_sc_gather():
  return pl.kernel(
    _sc_gather_body,
    mesh=plsc.VectorSubcoreMesh(core_axis_name="c", subcore_axis_name="s"),
    out_type=[
        jax.ShapeDtypeStruct((SEQ_ROWS, D), jnp.float32),
        jax.ShapeDtypeStruct((SPARSE_ROWS, D), jnp.float32),
        jax.ShapeDtypeStruct((ITEM_PAD, D), jnp.float32),
    ],
    scratch_types=[
        pltpu.VMEM((SEQ_CHUNK,), jnp.int32),
        pltpu.VMEM((SEQ_CHUNK, D), jnp.float32),
        pltpu.VMEM((SPARSE_PER_W,), jnp.int32),
        pltpu.VMEM((SPARSE_PER_W, D), jnp.float32),
        pltpu.VMEM((ITEM_PAD,), jnp.int32),
        pltpu.VMEM((ITEM_PAD, D), jnp.float32),
        pltpu.SemaphoreType.DMA,
    ],
    compiler_params=pltpu.CompilerParams(use_tc_tiling_on_sc=False),
  )


BB = 256
GB = BB // GQ
G = B // BB

IW1, IW2, IW3, IF1M, IF2M, IF1S, IF2S = 0, 1, 2, 3, 4, 5, 15


def _tc_dense_body(seq_ref, dense_ref, sparse_ref, item_ref, sel_ref,
                   wpk_ref, wd_ref, wb_ref, out_ref):
    ri = lax.broadcasted_iota(jnp.int32, (128, 128), 0) // D
    ci = lax.broadcasted_iota(jnp.int32, (128, 128), 1) // D
    gones = (ri == ci).astype(jnp.float32)

    S = seq_ref[...]
    m_s = jnp.mean(S, axis=0)
    m_t = S[MAXLEN - 1]
    c = m_s @ wpk_ref[IW2] + m_t @ wpk_ref[IW3] + wb_ref[1:2]
    S2 = S.reshape(MAXLEN * GB, 128)
    E = (S2 @ wpk_ref[IW1]).reshape(MAXLEN, GB, 128) + c[None, :, :]
    att = jax.nn.sigmoid(E)
    aw = att * wb_ref[0:1][None, :, :]
    alpha = (aw.reshape(MAXLEN * GB, 128) @ gones).reshape(MAXLEN, GB, 128)
    m_a = jnp.sum(alpha * S, axis=0)

    xd = dense_ref[...]
    acc1 = m_a @ wpk_ref[IF1M] + xd @ wd_ref[0] + wb_ref[2:3]
    acc2 = m_t @ wpk_ref[IF2M] + xd @ wd_ref[1] + wb_ref[3:4]
    for i in range(N_SPARSE):
        xi = sparse_ref[i]
        acc1 = acc1 + xi @ wpk_ref[IF1S + i]
        acc2 = acc2 + xi @ wpk_ref[IF2S + i]
    p4 = jnp.tanh(acc1) * jnp.tanh(acc2)

    p = sel_ref[0] @ p4[:, 0:D]
    for q in range(1, GQ):
        p = p + sel_ref[q] @ p4[:, q * D:(q + 1) * D]
    z = lax.dot_general(p, item_ref[...], (((1,), (1,)), ((), ())))
    z = z[:, :M_ITEMS]
    z = z - jnp.max(z, axis=-1, keepdims=True)
    ez = jnp.exp(z)
    out_ref[...] = ez / jnp.sum(ez, axis=-1, keepdims=True)


def _full_spec(shape):
    return pl.BlockSpec(shape, lambda i: tuple(0 for _ in shape))


_TC_IN_SPECS = [
        pl.BlockSpec((MAXLEN, GB, 128), lambda i: (0, i, 0)),
        pl.BlockSpec((GB, GQ * DENSE), lambda i: (i, 0)),
        pl.BlockSpec((N_SPARSE, GB, 128), lambda i: (0, i, 0)),
        _full_spec((ITEM_PAD, D)),
        _full_spec((GQ, BB, GB)),
        _full_spec((25, 128, 128)),
        _full_spec((2, GQ * DENSE, 128)),
        _full_spec((GQ, 128)),
]

_tc_dense = pl.pallas_call(
    _tc_dense_body,
    grid=(G,),
    in_specs=_TC_IN_SPECS,
    out_specs=pl.BlockSpec((BB, M_ITEMS), lambda i: (i, 0)),
    out_shape=jax.ShapeDtypeStruct((B, M_ITEMS), jnp.float32),
)


@jax.jit
def kernel(dense_inputs, sparse_inputs, seq_inputs, item_pooling, table_sparse,
           table_seq, W0, W1, W2, W3, b, ffn1_W, ffn1_b, ffn2_W, ffn2_b):
    tseqT = table_seq.T
    tspT = jnp.transpose(table_sparse, (0, 2, 1)).reshape(
        N_SPARSE * D, SPARSE_VOCAB)
    tseq_g = _transpose_seq(tseqT, tseqT, tseqT, tseqT).reshape(-1, D)
    tsp_g = _transpose_sp(tspT, tspT, tspT, tspT).reshape(-1, D)

    fseq = seq_inputs[:, 0, :].T
    fsp = sparse_inputs.T
    sp_base = (jnp.arange(N_SPARSE, dtype=jnp.int32) * QS_SP)[:, None]
    fit = jnp.concatenate(
        [item_pooling[:, 0], jnp.zeros((ITEM_PAD - M_ITEMS,), jnp.int32)])

    seq_flat, sparse_flat, item_embed = _get_sc_gather()(
        (LINES * (fseq % QS_SEQ) + fseq // QS_SEQ).reshape(-1),
        (LINES * (fsp % QS_SP + sp_base) + fsp // QS_SP).reshape(-1),
        LINES * (fit % QS_SEQ) + fit // QS_SEQ,
        tseq_g, tsp_g)

    seq_pk = seq_flat.reshape(MAXLEN, BL, 128)
    sparse_pk = sparse_flat.reshape(N_SPARSE, BL, 128)
    dense_pk = dense_inputs.reshape(BL, GQ * DENSE)

    wsm = jnp.concatenate([
        W1[None], W2[None], W3[None], ffn1_W[None, :D], ffn2_W[None, :D],
        ffn1_W[D + DENSE:].reshape(N_SPARSE, D, D),
        ffn2_W[D + DENSE:].reshape(N_SPARSE, D, D)])
    ri = jnp.arange(GQ * D)[:, None] // D
    ci = jnp.arange(GQ * D)[None, :] // D
    wpk = jnp.tile(wsm, (1, GQ, GQ)) * (ri == ci)[None]
    wdsm = jnp.stack([ffn1_W[D:D + DENSE], ffn2_W[D:D + DENSE]])
    rd = jnp.arange(GQ * DENSE)[:, None] // DENSE
    cd = jnp.arange(GQ * D)[None, :] // D
    wd = jnp.tile(wdsm, (1, GQ, GQ)) * (rd == cd)[None]
    wb = jnp.tile(jnp.stack([W0[:, 0], b, ffn1_b, ffn2_b]), (1, GQ))
    rows = jnp.arange(BB)
    cols = jnp.arange(GB)
    sel = jnp.stack([
        (rows[:, None] == cols[None, :] * GQ + q).astype(jnp.float32)
        for q in range(GQ)])

    return _tc_dense(seq_pk, dense_pk, sparse_pk, item_embed, sel,
                     wpk, wd, wb)


# speedup vs baseline: 2.3788x; 1.1306x over previous
import functools

import jax
import jax.numpy as jnp
from jax import lax
from jax.experimental import pallas as pl
from jax.experimental.pallas import tpu as pltpu
from jax.experimental.pallas import tpu_sc as plsc

B = 4096
DENSE = 8
N_SPARSE = 10
SPARSE_VOCAB = 100000
SEQ_VOCAB = 1000000
D = 32
MAXLEN = 50
M_ITEMS = 100

NC = 2
NS = 16
NW = NC * NS

SEQ_ROWS = B * MAXLEN
SEQ_PER_W = SEQ_ROWS // NW
SEQ_CHUNK = 1600
SPARSE_ROWS = B * N_SPARSE
SPARSE_PER_W = SPARSE_ROWS // NW
ITEM_PAD = 128
LINES = 4

QS_SEQ = 262144
QS_SP = 32768
TBLK = 8192
SEQ_LASTB = (SEQ_VOCAB + TBLK - 1) // TBLK - 1
SP_LASTB = (SPARSE_VOCAB + TBLK - 1) // TBLK - 1

GQ = 4
BL = B // GQ


def _transpose_seq_body(t0, t1, t2, t3, out_ref):
    x = jnp.concatenate([t0[...], t1[...], t2[...], t3[...]], axis=0)
    out_ref[...] = jnp.transpose(x)


_transpose_seq = pl.pallas_call(
    _transpose_seq_body,
    grid=(QS_SEQ // TBLK,),
    in_specs=[
        pl.BlockSpec(
            (D, TBLK),
            lambda i, q=q: (0, jnp.minimum(i + q * (QS_SEQ // TBLK),
                                           SEQ_LASTB)))
        for q in range(LINES)
    ],
    out_specs=pl.BlockSpec((TBLK, 128), lambda i: (i, 0)),
    out_shape=jax.ShapeDtypeStruct((QS_SEQ, 128), jnp.float32),
)


def _transpose_sp_body(t0, t1, t2, t3, out_ref):
    x = jnp.concatenate([t0[...], t1[...], t2[...], t3[...]], axis=0)
    out_ref[...] = jnp.transpose(x)


_transpose_sp = pl.pallas_call(
    _transpose_sp_body,
    grid=(N_SPARSE, QS_SP // TBLK),
    in_specs=[
        pl.BlockSpec(
            (D, TBLK),
            lambda i, j, q=q: (i, jnp.minimum(j + q * (QS_SP // TBLK),
                                              SP_LASTB)))
        for q in range(LINES)
    ],
    out_specs=pl.BlockSpec((TBLK, 128),
                           lambda i, j: (i * (QS_SP // TBLK) + j, 0)),
    out_shape=jax.ShapeDtypeStruct((N_SPARSE * QS_SP, 128), jnp.float32),
)


def _sc_gather_body(seq_idx, sparse_idx, item_idx, table_seq, table_sparse,
                    seq_out, sparse_out, item_out,
                    idx_v, rows_v, sidx_v, srows_v, iidx_v, irows_v, sem):
    wid = lax.axis_index("s") * NC + lax.axis_index("c")
    base = wid * SEQ_PER_W
    for ci in range(SEQ_PER_W // SEQ_CHUNK):
        off = base + ci * SEQ_CHUNK
        pltpu.sync_copy(seq_idx.at[pl.ds(off, SEQ_CHUNK)], idx_v)
        pltpu.async_copy(table_seq.at[idx_v], rows_v, sem).wait()
        pltpu.sync_copy(rows_v, seq_out.at[pl.ds(off, SEQ_CHUNK)])
    sbase = wid * SPARSE_PER_W
    pltpu.sync_copy(sparse_idx.at[pl.ds(sbase, SPARSE_PER_W)], sidx_v)
    pltpu.async_copy(table_sparse.at[sidx_v], srows_v, sem).wait()
    pltpu.sync_copy(srows_v, sparse_out.at[pl.ds(sbase, SPARSE_PER_W)])
    @pl.when(wid == 0)
    def _():
        pltpu.sync_copy(item_idx.at[pl.ds(0, ITEM_PAD)], iidx_v)
        pltpu.async_copy(table_seq.at[iidx_v], irows_v, sem).wait()
        pltpu.sync_copy(irows_v, item_out.at[pl.ds(0, ITEM_PAD)])


@functools.lru_cache(maxsize=1)
def _get_sc_gather():
  return pl.kernel(
    _sc_gather_body,
    mesh=plsc.VectorSubcoreMesh(core_axis_name="c", subcore_axis_name="s"),
    out_type=[
        jax.ShapeDtypeStruct((SEQ_ROWS, D), jnp.float32),
        jax.ShapeDtypeStruct((SPARSE_ROWS, D), jnp.float32),
        jax.ShapeDtypeStruct((ITEM_PAD, D), jnp.float32),
    ],
    scratch_types=[
        pltpu.VMEM((SEQ_CHUNK,), jnp.int32),
        pltpu.VMEM((SEQ_CHUNK, D), jnp.float32),
        pltpu.VMEM((SPARSE_PER_W,), jnp.int32),
        pltpu.VMEM((SPARSE_PER_W, D), jnp.float32),
        pltpu.VMEM((ITEM_PAD,), jnp.int32),
        pltpu.VMEM((ITEM_PAD, D), jnp.float32),
        pltpu.SemaphoreType.DMA,
    ],
    compiler_params=pltpu.CompilerParams(use_tc_tiling_on_sc=False),
  )


BB = 256
GB = BB // GQ
G = B // BB

IW1, IW2, IW3, IF1M, IF2M, IF1S, IF2S = 0, 1, 2, 3, 4, 5, 15


def _tc_dense_body(seq_ref, dense_ref, sparse_ref, item_ref, sel_ref,
                   wpk_ref, wd_ref, wb_ref, out_ref):
    ri = lax.broadcasted_iota(jnp.int32, (128, 128), 0) // D
    ci = lax.broadcasted_iota(jnp.int32, (128, 128), 1) // D
    gones = (ri == ci).astype(jnp.float32)

    S = seq_ref[...]
    m_s = jnp.mean(S, axis=0)
    m_t = S[MAXLEN - 1]
    c = m_s @ wpk_ref[IW2] + m_t @ wpk_ref[IW3] + wb_ref[1:2]
    S2 = S.reshape(MAXLEN * GB, 128)
    E = (S2 @ wpk_ref[IW1]).reshape(MAXLEN, GB, 128) + c[None, :, :]
    att = jax.nn.sigmoid(E)
    aw = att * wb_ref[0:1][None, :, :]
    alpha = (aw.reshape(MAXLEN * GB, 128) @ gones).reshape(MAXLEN, GB, 128)
    m_a = jnp.sum(alpha * S, axis=0)

    xd = dense_ref[...]
    acc1 = m_a @ wpk_ref[IF1M] + xd @ wd_ref[0] + wb_ref[2:3]
    acc2 = m_t @ wpk_ref[IF2M] + xd @ wd_ref[1] + wb_ref[3:4]
    for i in range(N_SPARSE):
        xi = sparse_ref[i]
        acc1 = acc1 + xi @ wpk_ref[IF1S + i]
        acc2 = acc2 + xi @ wpk_ref[IF2S + i]
    p4 = jnp.tanh(acc1) * jnp.tanh(acc2)

    p = sel_ref[0] @ p4[:, 0:D]
    for q in range(1, GQ):
        p = p + sel_ref[q] @ p4[:, q * D:(q + 1) * D]
    z = lax.dot_general(p, item_ref[...], (((1,), (1,)), ((), ())))
    z = z[:, :M_ITEMS]
    z = z - jnp.max(z, axis=-1, keepdims=True)
    ez = jnp.exp(z)
    out_ref[...] = ez / jnp.sum(ez, axis=-1, keepdims=True)


def _full_spec(shape):
    return pl.BlockSpec(shape, lambda i: tuple(0 for _ in shape))


_TC_IN_SPECS = [
        pl.BlockSpec((MAXLEN, GB, 128), lambda i: (0, i, 0)),
        pl.BlockSpec((GB, GQ * DENSE), lambda i: (i, 0)),
        pl.BlockSpec((N_SPARSE, GB, 128), lambda i: (0, i, 0)),
        _full_spec((ITEM_PAD, D)),
        _full_spec((GQ, BB, GB)),
        _full_spec((25, 128, 128)),
        _full_spec((2, GQ * DENSE, 128)),
        _full_spec((GQ, 128)),
]

_tc_dense = pl.pallas_call(
    _tc_dense_body,
    grid=(G,),
    in_specs=_TC_IN_SPECS,
    out_specs=pl.BlockSpec((BB, M_ITEMS), lambda i: (i, 0)),
    out_shape=jax.ShapeDtypeStruct((B, M_ITEMS), jnp.float32),
)


@jax.jit
def kernel(dense_inputs, sparse_inputs, seq_inputs, item_pooling, table_sparse,
           table_seq, W0, W1, W2, W3, b, ffn1_W, ffn1_b, ffn2_W, ffn2_b):
    tseqT = table_seq.T
    tspT = jnp.transpose(table_sparse, (0, 2, 1)).reshape(
        N_SPARSE * D, SPARSE_VOCAB)
    tseq_g = _transpose_seq(tseqT, tseqT, tseqT, tseqT).reshape(-1, D)
    tsp_g = _transpose_sp(tspT, tspT, tspT, tspT).reshape(-1, D)

    fseq = seq_inputs[:, 0, :].T
    fsp = sparse_inputs.T
    sp_base = (jnp.arange(N_SPARSE, dtype=jnp.int32) * QS_SP)[:, None]
    fit = jnp.concatenate(
        [item_pooling[:, 0], jnp.zeros((ITEM_PAD - M_ITEMS,), jnp.int32)])

    seq_flat, sparse_flat, item_embed = _get_sc_gather()(
        (LINES * (fseq % QS_SEQ) + fseq // QS_SEQ).reshape(-1),
        (LINES * (fsp % QS_SP + sp_base) + fsp // QS_SP).reshape(-1),
        LINES * (fit % QS_SEQ) + fit // QS_SEQ,
        tseq_g, tsp_g)

    seq_pk = seq_flat.reshape(MAXLEN, BL, 128)
    sparse_pk = sparse_flat.reshape(N_SPARSE, BL, 128)
    dense_pk = dense_inputs.reshape(BL, GQ * DENSE)

    wsm = jnp.concatenate([
        W1[None], W2[None], W3[None], ffn1_W[None, :D], ffn2_W[None, :D],
        ffn1_W[D + DENSE:].reshape(N_SPARSE, D, D),
        ffn2_W[D + DENSE:].reshape(N_SPARSE, D, D)])
    ri = jnp.arange(GQ * D)[:, None] // D
    ci = jnp.arange(GQ * D)[None, :] // D
    wpk = jnp.tile(wsm, (1, GQ, GQ)) * (ri == ci)[None]
    wdsm = jnp.stack([ffn1_W[D:D + DENSE], ffn2_W[D:D + DENSE]])
    rd = jnp.arange(GQ * DENSE)[:, None] // DENSE
    cd = jnp.arange(GQ * D)[None, :] // D
    wd = jnp.tile(wdsm, (1, GQ, GQ)) * (rd == cd)[None]
    wb = jnp.tile(jnp.stack([W0[:, 0], b, ffn1_b, ffn2_b]), (1, GQ))
    rows = jnp.arange(BB)
    cols = jnp.arange(GB)
    sel = jnp.stack([
        (rows[:, None] == cols[None, :] * GQ + q).astype(jnp.float32)
        for q in range(GQ)])

    return _tc_dense(seq_pk, dense_pk, sparse_pk, item_embed, sel,
                     wpk, wd, wb)


# speedup vs baseline: 2.4414x; 1.0263x over previous
import functools

import jax
import jax.numpy as jnp
from jax import lax
from jax.experimental import pallas as pl
from jax.experimental.pallas import tpu as pltpu
from jax.experimental.pallas import tpu_sc as plsc

B = 4096
DENSE = 8
N_SPARSE = 10
SPARSE_VOCAB = 100000
SEQ_VOCAB = 1000000
D = 32
MAXLEN = 50
M_ITEMS = 100

NC = 2
NS = 16
NW = NC * NS

SEQ_ROWS = B * MAXLEN
SEQ_PER_W = SEQ_ROWS // NW
SEQ_CHUNK = 1600
SPARSE_ROWS = B * N_SPARSE
SPARSE_PER_W = SPARSE_ROWS // NW
ITEM_PAD = 128
LINES = 4

QS_SEQ = 262144
QS_SP = 32768
TBLK = 16384
SEQ_LASTB = (SEQ_VOCAB + TBLK - 1) // TBLK - 1
SP_LASTB = (SPARSE_VOCAB + TBLK - 1) // TBLK - 1

GQ = 4
BL = B // GQ


def _transpose_seq_body(t0, t1, t2, t3, out_ref):
    x = jnp.concatenate([t0[...], t1[...], t2[...], t3[...]], axis=0)
    out_ref[...] = jnp.transpose(x)


_transpose_seq = pl.pallas_call(
    _transpose_seq_body,
    grid=(QS_SEQ // TBLK,),
    in_specs=[
        pl.BlockSpec(
            (D, TBLK),
            lambda i, q=q: (0, jnp.minimum(i + q * (QS_SEQ // TBLK),
                                           SEQ_LASTB)))
        for q in range(LINES)
    ],
    out_specs=pl.BlockSpec((TBLK, 128), lambda i: (i, 0)),
    out_shape=jax.ShapeDtypeStruct((QS_SEQ, 128), jnp.float32),
)


def _transpose_sp_body(t0, t1, t2, t3, out_ref):
    x = jnp.concatenate([t0[...], t1[...], t2[...], t3[...]], axis=0)
    out_ref[...] = jnp.transpose(x)


_transpose_sp = pl.pallas_call(
    _transpose_sp_body,
    grid=(N_SPARSE, QS_SP // TBLK),
    in_specs=[
        pl.BlockSpec(
            (D, TBLK),
            lambda i, j, q=q: (i, jnp.minimum(j + q * (QS_SP // TBLK),
                                              SP_LASTB)))
        for q in range(LINES)
    ],
    out_specs=pl.BlockSpec((TBLK, 128),
                           lambda i, j: (i * (QS_SP // TBLK) + j, 0)),
    out_shape=jax.ShapeDtypeStruct((N_SPARSE * QS_SP, 128), jnp.float32),
)


def _sc_gather_body(seq_idx, sparse_idx, item_idx, table_seq, table_sparse,
                    seq_out, sparse_out, item_out,
                    idx_v, rows_v, sidx_v, srows_v, iidx_v, irows_v, sem):
    wid = lax.axis_index("s") * NC + lax.axis_index("c")
    base = wid * SEQ_PER_W
    for ci in range(SEQ_PER_W // SEQ_CHUNK):
        off = base + ci * SEQ_CHUNK
        pltpu.sync_copy(seq_idx.at[pl.ds(off, SEQ_CHUNK)], idx_v)
        pltpu.async_copy(table_seq.at[idx_v], rows_v, sem).wait()
        pltpu.sync_copy(rows_v, seq_out.at[pl.ds(off, SEQ_CHUNK)])
    sbase = wid * SPARSE_PER_W
    pltpu.sync_copy(sparse_idx.at[pl.ds(sbase, SPARSE_PER_W)], sidx_v)
    pltpu.async_copy(table_sparse.at[sidx_v], srows_v, sem).wait()
    pltpu.sync_copy(srows_v, sparse_out.at[pl.ds(sbase, SPARSE_PER_W)])
    @pl.when(wid == 0)
    def _():
        pltpu.sync_copy(item_idx.at[pl.ds(0, ITEM_PAD)], iidx_v)
        pltpu.async_copy(table_seq.at[iidx_v], irows_v, sem).wait()
        pltpu.sync_copy(irows_v, item_out.at[pl.ds(0, ITEM_PAD)])


@functools.lru_cache(maxsize=1)
def _get_sc_gather():
  return pl.kernel(
    _sc_gather_body,
    mesh=plsc.VectorSubcoreMesh(core_axis_name="c", subcore_axis_name="s"),
    out_type=[
        jax.ShapeDtypeStruct((SEQ_ROWS, D), jnp.float32),
        jax.ShapeDtypeStruct((SPARSE_ROWS, D), jnp.float32),
        jax.ShapeDtypeStruct((ITEM_PAD, D), jnp.float32),
    ],
    scratch_types=[
        pltpu.VMEM((SEQ_CHUNK,), jnp.int32),
        pltpu.VMEM((SEQ_CHUNK, D), jnp.float32),
        pltpu.VMEM((SPARSE_PER_W,), jnp.int32),
        pltpu.VMEM((SPARSE_PER_W, D), jnp.float32),
        pltpu.VMEM((ITEM_PAD,), jnp.int32),
        pltpu.VMEM((ITEM_PAD, D), jnp.float32),
        pltpu.SemaphoreType.DMA,
    ],
    compiler_params=pltpu.CompilerParams(use_tc_tiling_on_sc=False),
  )


BB = 256
GB = BB // GQ
G = B // BB

IW1, IW2, IW3, IF1M, IF2M, IF1S, IF2S = 0, 1, 2, 3, 4, 5, 15


def _tc_dense_body(seq_ref, dense_ref, sparse_ref, item_ref, sel_ref,
                   wpk_ref, wd_ref, wb_ref, out_ref):
    ri = lax.broadcasted_iota(jnp.int32, (128, 128), 0) // D
    ci = lax.broadcasted_iota(jnp.int32, (128, 128), 1) // D
    gones = (ri == ci).astype(jnp.float32)

    S = seq_ref[...]
    m_s = jnp.mean(S, axis=0)
    m_t = S[MAXLEN - 1]
    c = m_s @ wpk_ref[IW2] + m_t @ wpk_ref[IW3] + wb_ref[1:2]
    S2 = S.reshape(MAXLEN * GB, 128)
    E = (S2 @ wpk_ref[IW1]).reshape(MAXLEN, GB, 128) + c[None, :, :]
    att = jax.nn.sigmoid(E)
    aw = att * wb_ref[0:1][None, :, :]
    alpha = (aw.reshape(MAXLEN * GB, 128) @ gones).reshape(MAXLEN, GB, 128)
    m_a = jnp.sum(alpha * S, axis=0)

    xd = dense_ref[...]
    acc1 = m_a @ wpk_ref[IF1M] + xd @ wd_ref[0] + wb_ref[2:3]
    acc2 = m_t @ wpk_ref[IF2M] + xd @ wd_ref[1] + wb_ref[3:4]
    for i in range(N_SPARSE):
        xi = sparse_ref[i]
        acc1 = acc1 + xi @ wpk_ref[IF1S + i]
        acc2 = acc2 + xi @ wpk_ref[IF2S + i]
    p4 = jnp.tanh(acc1) * jnp.tanh(acc2)

    p = sel_ref[0] @ p4[:, 0:D]
    for q in range(1, GQ):
        p = p + sel_ref[q] @ p4[:, q * D:(q + 1) * D]
    z = lax.dot_general(p, item_ref[...], (((1,), (1,)), ((), ())))
    z = z[:, :M_ITEMS]
    z = z - jnp.max(z, axis=-1, keepdims=True)
    ez = jnp.exp(z)
    out_ref[...] = ez / jnp.sum(ez, axis=-1, keepdims=True)


def _full_spec(shape):
    return pl.BlockSpec(shape, lambda i: tuple(0 for _ in shape))


_TC_IN_SPECS = [
        pl.BlockSpec((MAXLEN, GB, 128), lambda i: (0, i, 0)),
        pl.BlockSpec((GB, GQ * DENSE), lambda i: (i, 0)),
        pl.BlockSpec((N_SPARSE, GB, 128), lambda i: (0, i, 0)),
        _full_spec((ITEM_PAD, D)),
        _full_spec((GQ, BB, GB)),
        _full_spec((25, 128, 128)),
        _full_spec((2, GQ * DENSE, 128)),
        _full_spec((GQ, 128)),
]

_tc_dense = pl.pallas_call(
    _tc_dense_body,
    grid=(G,),
    in_specs=_TC_IN_SPECS,
    out_specs=pl.BlockSpec((BB, M_ITEMS), lambda i: (i, 0)),
    out_shape=jax.ShapeDtypeStruct((B, M_ITEMS), jnp.float32),
)


@jax.jit
def kernel(dense_inputs, sparse_inputs, seq_inputs, item_pooling, table_sparse,
           table_seq, W0, W1, W2, W3, b, ffn1_W, ffn1_b, ffn2_W, ffn2_b):
    tseqT = table_seq.T
    tspT = jnp.transpose(table_sparse, (0, 2, 1)).reshape(
        N_SPARSE * D, SPARSE_VOCAB)
    tseq_g = _transpose_seq(tseqT, tseqT, tseqT, tseqT).reshape(-1, D)
    tsp_g = _transpose_sp(tspT, tspT, tspT, tspT).reshape(-1, D)

    fseq = seq_inputs[:, 0, :].T
    fsp = sparse_inputs.T
    sp_base = (jnp.arange(N_SPARSE, dtype=jnp.int32) * QS_SP)[:, None]
    fit = jnp.concatenate(
        [item_pooling[:, 0], jnp.zeros((ITEM_PAD - M_ITEMS,), jnp.int32)])

    seq_flat, sparse_flat, item_embed = _get_sc_gather()(
        (LINES * (fseq % QS_SEQ) + fseq // QS_SEQ).reshape(-1),
        (LINES * (fsp % QS_SP + sp_base) + fsp // QS_SP).reshape(-1),
        LINES * (fit % QS_SEQ) + fit // QS_SEQ,
        tseq_g, tsp_g)

    seq_pk = seq_flat.reshape(MAXLEN, BL, 128)
    sparse_pk = sparse_flat.reshape(N_SPARSE, BL, 128)
    dense_pk = dense_inputs.reshape(BL, GQ * DENSE)

    wsm = jnp.concatenate([
        W1[None], W2[None], W3[None], ffn1_W[None, :D], ffn2_W[None, :D],
        ffn1_W[D + DENSE:].reshape(N_SPARSE, D, D),
        ffn2_W[D + DENSE:].reshape(N_SPARSE, D, D)])
    ri = jnp.arange(GQ * D)[:, None] // D
    ci = jnp.arange(GQ * D)[None, :] // D
    wpk = jnp.tile(wsm, (1, GQ, GQ)) * (ri == ci)[None]
    wdsm = jnp.stack([ffn1_W[D:D + DENSE], ffn2_W[D:D + DENSE]])
    rd = jnp.arange(GQ * DENSE)[:, None] // DENSE
    cd = jnp.arange(GQ * D)[None, :] // D
    wd = jnp.tile(wdsm, (1, GQ, GQ)) * (rd == cd)[None]
    wb = jnp.tile(jnp.stack([W0[:, 0], b, ffn1_b, ffn2_b]), (1, GQ))
    rows = jnp.arange(BB)
    cols = jnp.arange(GB)
    sel = jnp.stack([
        (rows[:, None] == cols[None, :] * GQ + q).astype(jnp.float32)
        for q in range(GQ)])

    return _tc_dense(seq_pk, dense_pk, sparse_pk, item_embed, sel,
                     wpk, wd, wb)
